# Initial kernel scaffold; baseline (speedup 1.0000x reference)
import jax
import jax.numpy as jnp
from jax.experimental import pallas as pl


def kernel(head_idx, relation_idx, tail_idx, negative_idx, entity_emb, relation_emb):
    raise NotImplementedError("write your pallas kernel here")


# speedup vs baseline: 2.2418x; 2.2418x over previous
import functools

import jax
import jax.numpy as jnp
from jax import lax
from jax.experimental import pallas as pl
from jax.experimental.pallas import tpu as pltpu
from jax.experimental.pallas import tpu_sc as plsc

_NC = 2
_NS = 16
_NW = _NC * _NS
_CHUNK = 128


def _sc_gather_build(n_ent_rows, n_rel_rows, dim):
    ent_per_w = n_ent_rows // _NW
    rel_per_w = n_rel_rows // _NW
    n_chunks = ent_per_w // _CHUNK
    mesh = plsc.VectorSubcoreMesh(core_axis_name="c", subcore_axis_name="s")

    @functools.partial(
        pl.kernel,
        out_type=[
            jax.ShapeDtypeStruct((n_ent_rows, dim), jnp.float32),
            jax.ShapeDtypeStruct((n_rel_rows, dim), jnp.float32),
        ],
        mesh=mesh,
        scratch_types=[
            pltpu.VMEM((_CHUNK,), jnp.int32),
            pltpu.VMEM((_CHUNK, dim), jnp.float32),
            pltpu.VMEM((rel_per_w,), jnp.int32),
            pltpu.VMEM((rel_per_w, dim), jnp.float32),
            pltpu.SemaphoreType.DMA,
        ],
    )
    def k(ent_hbm, eidx_hbm, rel_hbm, ridx_hbm, ent_out, rel_out,
          idx_v, rows_v, ridx_v, rrows_v, sem):
        wid = lax.axis_index("s") * _NC + lax.axis_index("c")

        rbase = wid * rel_per_w
        pltpu.sync_copy(ridx_hbm.at[pl.ds(rbase, rel_per_w)], ridx_v)
        pltpu.async_copy(rel_hbm.at[ridx_v], rrows_v, sem).wait()
        pltpu.sync_copy(rrows_v, rel_out.at[pl.ds(rbase, rel_per_w)])

        def body(c, carry):
            base = wid * ent_per_w + c * _CHUNK
            pltpu.sync_copy(eidx_hbm.at[pl.ds(base, _CHUNK)], idx_v)
            pltpu.async_copy(ent_hbm.at[idx_v], rows_v, sem).wait()
            pltpu.sync_copy(rows_v, ent_out.at[pl.ds(base, _CHUNK)])
            return carry

        lax.fori_loop(0, n_chunks, body, 0)

    return k


def _tc_score_body(head_ref, rel_ref, tail_ref, neg_ref, pos_ref, negs_ref):
    eps = 1e-12
    h = head_ref[...]
    r = rel_ref[...]
    t = tail_ref[...]
    n = neg_ref[...]
    hn = h / jnp.maximum(jnp.sqrt(jnp.sum(h * h, axis=-1, keepdims=True)), eps)
    tn = t / jnp.maximum(jnp.sqrt(jnp.sum(t * t, axis=-1, keepdims=True)), eps)
    nn = n / jnp.maximum(jnp.sqrt(jnp.sum(n * n, axis=-1, keepdims=True)), eps)
    pd = hn + r - tn
    pos_ref[...] = -jnp.sqrt(jnp.sum(pd * pd, axis=-1))
    nd = hn[:, None, :] + r[:, None, :] - nn
    negs_ref[...] = -jnp.sqrt(jnp.sum(nd * nd, axis=-1))


def _tc_score(head, rel, tail, neg):
    batch, dim = head.shape
    n_neg = neg.shape[1]
    blk = 256
    grid = (batch // blk,)
    return pl.pallas_call(
        _tc_score_body,
        grid=grid,
        in_specs=[
            pl.BlockSpec((blk, dim), lambda i: (i, 0)),
            pl.BlockSpec((blk, dim), lambda i: (i, 0)),
            pl.BlockSpec((blk, dim), lambda i: (i, 0)),
            pl.BlockSpec((blk, n_neg, dim), lambda i: (i, 0, 0)),
        ],
        out_specs=[
            pl.BlockSpec((blk,), lambda i: (i,)),
            pl.BlockSpec((blk, n_neg), lambda i: (i, 0)),
        ],
        out_shape=[
            jax.ShapeDtypeStruct((batch,), jnp.float32),
            jax.ShapeDtypeStruct((batch, n_neg), jnp.float32),
        ],
    )(head, rel, tail, neg)


def kernel(head_idx, relation_idx, tail_idx, negative_idx, entity_emb, relation_emb):
    batch = head_idx.shape[0]
    n_neg = negative_idx.shape[1]
    dim = entity_emb.shape[1]
    all_idx = jnp.concatenate([head_idx, tail_idx, negative_idx.reshape(-1)])
    sc = _sc_gather_build(all_idx.shape[0], batch, dim)
    ent_rows, rel_rows = sc(entity_emb, all_idx, relation_emb, relation_idx)
    head = ent_rows[:batch]
    tail = ent_rows[batch:2 * batch]
    neg = ent_rows[2 * batch:].reshape(batch, n_neg, dim)
    return _tc_score(head, rel_rows, tail, neg)


# speedup vs baseline: 3.3841x; 1.5095x over previous
import functools

import jax
import jax.numpy as jnp
from jax import lax
from jax.experimental import pallas as pl
from jax.experimental.pallas import tpu as pltpu
from jax.experimental.pallas import tpu_sc as plsc

_NC = 2
_NS = 16
_NW = _NC * _NS
_L = 16
_ROWS = 64
_CPE = 2
_CHUNK = _ROWS * _CPE


def _tree_sum(parts):
    while len(parts) > 1:
        parts = [a + b for a, b in zip(parts[::2], parts[1::2])]
    return parts[0]


def _lane_sum(x, perms):
    dn = lax.GatherDimensionNumbers(
        offset_dims=(), collapsed_slice_dims=(0,), start_index_map=(0,))
    for p in perms:
        x = x + lax.gather(x, p[:, None], dn, slice_sizes=(1,),
                           mode=lax.GatherScatterMode.PROMISE_IN_BOUNDS)
    return x


def _sc_build(batch, dim):
    per_w = batch // _NW
    n_chunks = per_w // _CPE
    dc = dim // _L
    mesh = plsc.VectorSubcoreMesh(core_axis_name="c", subcore_axis_name="s")

    @functools.partial(
        pl.kernel,
        out_type=[
            jax.ShapeDtypeStruct((batch * _ROWS,), jnp.float32),
            jax.ShapeDtypeStruct((batch * _ROWS,), jnp.float32),
            jax.ShapeDtypeStruct((batch * _ROWS,), jnp.float32),
            jax.ShapeDtypeStruct((batch * _L,), jnp.float32),
            jax.ShapeDtypeStruct((batch * _L,), jnp.float32),
        ],
        mesh=mesh,
        scratch_types=[
            pltpu.VMEM((per_w,), jnp.int32),
            pltpu.VMEM((per_w,), jnp.int32),
            pltpu.VMEM((per_w, dim), jnp.float32),
            pltpu.VMEM((per_w, dim), jnp.float32),
            pltpu.VMEM((per_w * _L,), jnp.float32),
            pltpu.VMEM((per_w * _L,), jnp.float32),
            pltpu.VMEM((_CHUNK,), jnp.int32),
            pltpu.VMEM((_CHUNK, dim), jnp.float32),
            pltpu.VMEM((_CHUNK,), jnp.float32),
            pltpu.VMEM((_CHUNK,), jnp.float32),
            pltpu.VMEM((_CHUNK,), jnp.float32),
            pltpu.SemaphoreType.DMA,
        ],
    )
    def k(ent_hbm, hidx_hbm, rel_hbm, ridx_hbm, gidx_hbm,
          a_out, b_out, ss_out, hh_out, hr_out,
          hidx_v, ridx_v, hrows_v, rrows_v, hh_v, hr_v,
          gidx_v, rows_v, a_v, b_v, ss_v, sem):
        wid = lax.axis_index("s") * _NC + lax.axis_index("c")
        ebase = wid * per_w
        lanes = lax.iota(jnp.int32, _L)
        perms = [lanes ^ k for k in (8, 4, 2, 1)]

        pltpu.sync_copy(hidx_hbm.at[pl.ds(ebase, per_w)], hidx_v)
        pltpu.sync_copy(ridx_hbm.at[pl.ds(ebase, per_w)], ridx_v)
        pltpu.async_copy(ent_hbm.at[hidx_v], hrows_v, sem).wait()
        pltpu.async_copy(rel_hbm.at[ridx_v], rrows_v, sem).wait()

        def stage_a(e, carry):
            hch = [hrows_v[e, pl.ds(c * _L, _L)] for c in range(dc)]
            rch = [rrows_v[e, pl.ds(c * _L, _L)] for c in range(dc)]
            hh = _lane_sum(_tree_sum([h * h for h in hch]), perms)
            hr = _lane_sum(_tree_sum([h * r for h, r in zip(hch, rch)]),
                           perms)
            hh_v[pl.ds(e * _L, _L)] = hh
            hr_v[pl.ds(e * _L, _L)] = hr
            return carry

        lax.fori_loop(0, per_w, stage_a, 0)
        pltpu.sync_copy(hh_v, hh_out.at[pl.ds(ebase * _L, per_w * _L)])
        pltpu.sync_copy(hr_v, hr_out.at[pl.ds(ebase * _L, per_w * _L)])

        def stage_b(ch, carry):
            rbase = (ebase + ch * _CPE) * _ROWS
            pltpu.sync_copy(gidx_hbm.at[pl.ds(rbase, _CHUNK)], gidx_v)
            pltpu.async_copy(ent_hbm.at[gidx_v], rows_v, sem).wait()
            for sub in range(_CPE):
                e = ch * _CPE + sub
                hch = [hrows_v[e, pl.ds(c * _L, _L)] for c in range(dc)]
                rch = [rrows_v[e, pl.ds(c * _L, _L)] for c in range(dc)]

                def group_body(g, carry2):
                    res_a = jnp.zeros((_L,), jnp.float32)
                    res_b = jnp.zeros((_L,), jnp.float32)
                    res_ss = jnp.zeros((_L,), jnp.float32)
                    for l in range(_L):
                        row = sub * _ROWS + g * _L + l
                        nch = [rows_v[row, pl.ds(c * _L, _L)]
                               for c in range(dc)]
                        av = _lane_sum(
                            _tree_sum([n * h for n, h in zip(nch, hch)]),
                            perms)
                        bv = _lane_sum(
                            _tree_sum([n * r for n, r in zip(nch, rch)]),
                            perms)
                        ssv = _lane_sum(_tree_sum([n * n for n in nch]),
                                        perms)
                        res_a = jnp.where(lanes == l, av, res_a)
                        res_b = jnp.where(lanes == l, bv, res_b)
                        res_ss = jnp.where(lanes == l, ssv, res_ss)
                    off = sub * _ROWS + g * _L
                    a_v[pl.ds(off, _L)] = res_a
                    b_v[pl.ds(off, _L)] = res_b
                    ss_v[pl.ds(off, _L)] = res_ss
                    return carry2

                lax.fori_loop(0, _ROWS // _L, group_body, 0)
            pltpu.sync_copy(a_v, a_out.at[pl.ds(rbase, _CHUNK)])
            pltpu.sync_copy(b_v, b_out.at[pl.ds(rbase, _CHUNK)])
            pltpu.sync_copy(ss_v, ss_out.at[pl.ds(rbase, _CHUNK)])
            return carry

        lax.fori_loop(0, n_chunks, stage_b, 0)

    return k


def _tc_epilogue_body(a_ref, b_ref, ss_ref, hh_ref, hr_ref,
                      pos_ref, negs_ref):
    a = a_ref[...]
    b = b_ref[...]
    ss = ss_ref[...]
    hh = hh_ref[...][:, :1]
    hr = hr_ref[...][:, :1]
    rhh = 1.0 / jnp.maximum(jnp.sqrt(hh), 1e-12)
    qq = 2.0 + 2.0 * hr * rhh
    s = a * rhh + b
    rn = 1.0 / jnp.maximum(jnp.sqrt(ss), 1e-12)
    sc2 = jnp.maximum(qq + 1.0 - 2.0 * s * rn, 0.0)
    sc = -jnp.sqrt(sc2)
    pos_ref[...] = sc[:, 0]
    negs_ref[...] = sc[:, 1:51]


def _tc_epilogue(a, b, ss, hh, hr, batch, n_neg):
    blk = 512
    return pl.pallas_call(
        _tc_epilogue_body,
        grid=(batch // blk,),
        in_specs=[
            pl.BlockSpec((blk, _ROWS), lambda i: (i, 0)),
            pl.BlockSpec((blk, _ROWS), lambda i: (i, 0)),
            pl.BlockSpec((blk, _ROWS), lambda i: (i, 0)),
            pl.BlockSpec((blk, _L), lambda i: (i, 0)),
            pl.BlockSpec((blk, _L), lambda i: (i, 0)),
        ],
        out_specs=[
            pl.BlockSpec((blk,), lambda i: (i,)),
            pl.BlockSpec((blk, n_neg), lambda i: (i, 0)),
        ],
        out_shape=[
            jax.ShapeDtypeStruct((batch,), jnp.float32),
            jax.ShapeDtypeStruct((batch, n_neg), jnp.float32),
        ],
    )(a, b, ss, hh, hr)


def kernel(head_idx, relation_idx, tail_idx, negative_idx, entity_emb, relation_emb):
    batch = head_idx.shape[0]
    n_neg = negative_idx.shape[1]
    dim = entity_emb.shape[1]
    gidx = jnp.concatenate(
        [tail_idx[:, None], negative_idx, negative_idx[:, : _ROWS - 1 - n_neg]],
        axis=1).reshape(-1)
    sc = _sc_build(batch, dim)
    a, b, ss, hh, hr = sc(entity_emb, head_idx, relation_emb, relation_idx,
                          gidx)
    pos, negs = _tc_epilogue(
        a.reshape(batch, _ROWS), b.reshape(batch, _ROWS),
        ss.reshape(batch, _ROWS), hh.reshape(batch, _L),
        hr.reshape(batch, _L), batch, n_neg)
    return pos, negs


# speedup vs baseline: 4.9366x; 1.4587x over previous
import functools

import jax
import jax.numpy as jnp
from jax import lax
from jax.experimental import pallas as pl
from jax.experimental.pallas import tpu as pltpu
from jax.experimental.pallas import tpu_sc as plsc

_NC = 2
_NS = 16
_NW = _NC * _NS
_L = 16
_ROWS = 64
_CPE = 2
_CHUNK = _ROWS * _CPE


def _tree_sum(parts):
    while len(parts) > 1:
        parts = [a + b for a, b in zip(parts[::2], parts[1::2])]
    return parts[0]


def _lane_sum(x, perms):
    dn = lax.GatherDimensionNumbers(
        offset_dims=(), collapsed_slice_dims=(0,), start_index_map=(0,))
    for p in perms:
        x = x + lax.gather(x, p[:, None], dn, slice_sizes=(1,),
                           mode=lax.GatherScatterMode.PROMISE_IN_BOUNDS)
    return x


def _sc_build(batch, dim):
    per_w = batch // _NW
    n_chunks = per_w // _CPE
    dc = dim // _L
    mesh = plsc.VectorSubcoreMesh(core_axis_name="c", subcore_axis_name="s")

    @functools.partial(
        pl.kernel,
        out_type=[
            jax.ShapeDtypeStruct((batch * _ROWS,), jnp.float32),
            jax.ShapeDtypeStruct((batch * _ROWS,), jnp.float32),
            jax.ShapeDtypeStruct((batch * _ROWS,), jnp.float32),
            jax.ShapeDtypeStruct((batch * _L,), jnp.float32),
            jax.ShapeDtypeStruct((batch * _L,), jnp.float32),
        ],
        mesh=mesh,
        scratch_types=[
            pltpu.VMEM((per_w,), jnp.int32),
            pltpu.VMEM((per_w,), jnp.int32),
            pltpu.VMEM((per_w, dim), jnp.float32),
            pltpu.VMEM((per_w, dim), jnp.float32),
            pltpu.VMEM((per_w * _L,), jnp.float32),
            pltpu.VMEM((per_w * _L,), jnp.float32),
            pltpu.VMEM((per_w * _ROWS,), jnp.int32),
            pltpu.VMEM((2, _CHUNK, dim), jnp.float32),
            pltpu.VMEM((2, _CHUNK), jnp.float32),
            pltpu.VMEM((2, _CHUNK), jnp.float32),
            pltpu.VMEM((2, _CHUNK), jnp.float32),
            pltpu.SemaphoreType.DMA,
            pltpu.SemaphoreType.DMA,
            pltpu.SemaphoreType.DMA,
            pltpu.SemaphoreType.DMA,
            pltpu.SemaphoreType.DMA,
        ],
    )
    def k(ent_hbm, hidx_hbm, rel_hbm, ridx_hbm, gidx_hbm,
          a_out, b_out, ss_out, hh_out, hr_out,
          hidx_v, ridx_v, hrows_v, rrows_v, hh_v, hr_v,
          gidx_v, rows_v, a_v, b_v, ss_v, sem,
          gsem0, gsem1, osem0, osem1):
        wid = lax.axis_index("s") * _NC + lax.axis_index("c")
        ebase = wid * per_w
        lanes = lax.iota(jnp.int32, _L)
        perms = [lanes ^ k for k in (8, 4, 2, 1)]

        pltpu.sync_copy(hidx_hbm.at[pl.ds(ebase, per_w)], hidx_v)
        pltpu.sync_copy(ridx_hbm.at[pl.ds(ebase, per_w)], ridx_v)
        pltpu.async_copy(ent_hbm.at[hidx_v], hrows_v, sem).wait()
        pltpu.async_copy(rel_hbm.at[ridx_v], rrows_v, sem).wait()

        def stage_a(e, carry):
            hch = [hrows_v[e, pl.ds(c * _L, _L)] for c in range(dc)]
            rch = [rrows_v[e, pl.ds(c * _L, _L)] for c in range(dc)]
            hh = _lane_sum(_tree_sum([h * h for h in hch]), perms)
            hr = _lane_sum(_tree_sum([h * r for h, r in zip(hch, rch)]),
                           perms)
            hh_v[pl.ds(e * _L, _L)] = hh
            hr_v[pl.ds(e * _L, _L)] = hr
            return carry

        lax.fori_loop(0, per_w, stage_a, 0)
        pltpu.sync_copy(hh_v, hh_out.at[pl.ds(ebase * _L, per_w * _L)])
        pltpu.sync_copy(hr_v, hr_out.at[pl.ds(ebase * _L, per_w * _L)])

        pltpu.sync_copy(gidx_hbm.at[pl.ds(ebase * _ROWS, per_w * _ROWS)],
                        gidx_v)

        def gather_of(ch, p, gsem):
            return pltpu.async_copy(
                ent_hbm.at[gidx_v.at[pl.ds(ch * _CHUNK, _CHUNK)]],
                rows_v.at[p], gsem)

        def compute_chunk(ch, p):
            for sub in range(_CPE):
                e = ch * _CPE + sub
                hch = [hrows_v[e, pl.ds(c * _L, _L)] for c in range(dc)]
                rch = [rrows_v[e, pl.ds(c * _L, _L)] for c in range(dc)]

                def group_body(g, carry2):
                    res_a = jnp.zeros((_L,), jnp.float32)
                    res_b = jnp.zeros((_L,), jnp.float32)
                    res_ss = jnp.zeros((_L,), jnp.float32)
                    for l in range(_L):
                        row = sub * _ROWS + g * _L + l
                        nch = [rows_v[p, row, pl.ds(c * _L, _L)]
                               for c in range(dc)]
                        av = _lane_sum(
                            _tree_sum([n * h for n, h in zip(nch, hch)]),
                            perms)
                        bv = _lane_sum(
                            _tree_sum([n * r for n, r in zip(nch, rch)]),
                            perms)
                        ssv = _lane_sum(_tree_sum([n * n for n in nch]),
                                        perms)
                        res_a = jnp.where(lanes == l, av, res_a)
                        res_b = jnp.where(lanes == l, bv, res_b)
                        res_ss = jnp.where(lanes == l, ssv, res_ss)
                    off = sub * _ROWS + g * _L
                    a_v[p, pl.ds(off, _L)] = res_a
                    b_v[p, pl.ds(off, _L)] = res_b
                    ss_v[p, pl.ds(off, _L)] = res_ss
                    return carry2

                lax.fori_loop(0, _ROWS // _L, group_body, 0)

        def writeout(ch, p, osem):
            rbase = (ebase + ch * _CPE) * _ROWS
            pltpu.async_copy(a_v.at[p], a_out.at[pl.ds(rbase, _CHUNK)], osem)
            pltpu.async_copy(b_v.at[p], b_out.at[pl.ds(rbase, _CHUNK)], osem)
            pltpu.async_copy(ss_v.at[p], ss_out.at[pl.ds(rbase, _CHUNK)],
                             osem)

        def drain_out(ch, p, osem):
            rbase = (ebase + ch * _CPE) * _ROWS
            pltpu.make_async_copy(
                a_v.at[p], a_out.at[pl.ds(rbase, _CHUNK)], osem).wait()
            pltpu.make_async_copy(
                b_v.at[p], b_out.at[pl.ds(rbase, _CHUNK)], osem).wait()
            pltpu.make_async_copy(
                ss_v.at[p], ss_out.at[pl.ds(rbase, _CHUNK)], osem).wait()

        gather_of(0, 0, gsem0)

        def pair_body(i, carry):
            c0 = 2 * i
            pltpu.make_async_copy(
                ent_hbm.at[gidx_v.at[pl.ds(c0 * _CHUNK, _CHUNK)]],
                rows_v.at[0], gsem0).wait()
            gather_of(c0 + 1, 1, gsem1)

            @pl.when(i > 0)
            def _():
                drain_out(c0 - 2, 0, osem0)

            compute_chunk(c0, 0)
            writeout(c0, 0, osem0)

            pltpu.make_async_copy(
                ent_hbm.at[gidx_v.at[pl.ds((c0 + 1) * _CHUNK, _CHUNK)]],
                rows_v.at[1], gsem1).wait()
            gather_of((c0 + 2) % n_chunks, 0, gsem0)

            @pl.when(i > 0)
            def _():
                drain_out(c0 - 1, 1, osem1)

            compute_chunk(c0 + 1, 1)
            writeout(c0 + 1, 1, osem1)
            return carry

        lax.fori_loop(0, n_chunks // 2, pair_body, 0)
        pltpu.make_async_copy(
            ent_hbm.at[gidx_v.at[pl.ds(0, _CHUNK)]],
            rows_v.at[0], gsem0).wait()
        drain_out(n_chunks - 2, 0, osem0)
        drain_out(n_chunks - 1, 1, osem1)

    return k


def _tc_epilogue_body(a_ref, b_ref, ss_ref, hh_ref, hr_ref,
                      pos_ref, negs_ref):
    a = a_ref[...]
    b = b_ref[...]
    ss = ss_ref[...]
    hh = hh_ref[...][:, :1]
    hr = hr_ref[...][:, :1]
    rhh = 1.0 / jnp.maximum(jnp.sqrt(hh), 1e-12)
    qq = 2.0 + 2.0 * hr * rhh
    s = a * rhh + b
    rn = 1.0 / jnp.maximum(jnp.sqrt(ss), 1e-12)
    sc2 = jnp.maximum(qq + 1.0 - 2.0 * s * rn, 0.0)
    sc = -jnp.sqrt(sc2)
    pos_ref[...] = sc[:, 0]
    negs_ref[...] = sc[:, 1:51]


def _tc_epilogue(a, b, ss, hh, hr, batch, n_neg):
    blk = 512
    return pl.pallas_call(
        _tc_epilogue_body,
        grid=(batch // blk,),
        in_specs=[
            pl.BlockSpec((blk, _ROWS), lambda i: (i, 0)),
            pl.BlockSpec((blk, _ROWS), lambda i: (i, 0)),
            pl.BlockSpec((blk, _ROWS), lambda i: (i, 0)),
            pl.BlockSpec((blk, _L), lambda i: (i, 0)),
            pl.BlockSpec((blk, _L), lambda i: (i, 0)),
        ],
        out_specs=[
            pl.BlockSpec((blk,), lambda i: (i,)),
            pl.BlockSpec((blk, n_neg), lambda i: (i, 0)),
        ],
        out_shape=[
            jax.ShapeDtypeStruct((batch,), jnp.float32),
            jax.ShapeDtypeStruct((batch, n_neg), jnp.float32),
        ],
    )(a, b, ss, hh, hr)


def kernel(head_idx, relation_idx, tail_idx, negative_idx, entity_emb, relation_emb):
    batch = head_idx.shape[0]
    n_neg = negative_idx.shape[1]
    dim = entity_emb.shape[1]
    gidx = jnp.concatenate(
        [tail_idx[:, None], negative_idx, negative_idx[:, : _ROWS - 1 - n_neg]],
        axis=1).reshape(-1)
    sc = _sc_build(batch, dim)
    a, b, ss, hh, hr = sc(entity_emb, head_idx, relation_emb, relation_idx,
                          gidx)
    pos, negs = _tc_epilogue(
        a.reshape(batch, _ROWS), b.reshape(batch, _ROWS),
        ss.reshape(batch, _ROWS), hh.reshape(batch, _L),
        hr.reshape(batch, _L), batch, n_neg)
    return pos, negs


# speedup vs baseline: 6.9624x; 1.4104x over previous
import functools

import jax
import jax.numpy as jnp
from jax import lax
from jax.experimental import pallas as pl
from jax.experimental.pallas import tpu as pltpu
from jax.experimental.pallas import tpu_sc as plsc

_NC = 2
_NS = 16
_NW = _NC * _NS
_L = 16
_ROWS = 64
_CPE = 2
_CHUNK = _ROWS * _CPE


def _tree_sum(parts):
    while len(parts) > 1:
        parts = [a + b for a, b in zip(parts[::2], parts[1::2])]
    return parts[0]


def _lane_sum(x, perms):
    dn = lax.GatherDimensionNumbers(
        offset_dims=(), collapsed_slice_dims=(0,), start_index_map=(0,))
    for p in perms:
        x = x + lax.gather(x, p[:, None], dn, slice_sizes=(1,),
                           mode=lax.GatherScatterMode.PROMISE_IN_BOUNDS)
    return x


def _sc_build(batch, dim):
    per_w = batch // _NW
    n_chunks = per_w // _CPE
    dc = dim // _L
    mesh = plsc.VectorSubcoreMesh(core_axis_name="c", subcore_axis_name="s")

    @functools.partial(
        pl.kernel,
        out_type=[
            jax.ShapeDtypeStruct((batch * _ROWS,), jnp.float32),
            jax.ShapeDtypeStruct((batch * _ROWS,), jnp.float32),
            jax.ShapeDtypeStruct((batch * _L,), jnp.float32),
        ],
        mesh=mesh,
        scratch_types=[
            pltpu.VMEM((per_w,), jnp.int32),
            pltpu.VMEM((per_w,), jnp.int32),
            pltpu.VMEM((per_w, dim), jnp.float32),
            pltpu.VMEM((per_w, dim), jnp.float32),
            pltpu.VMEM((per_w, dim), jnp.float32),
            pltpu.VMEM((per_w * _L,), jnp.float32),
            pltpu.VMEM((per_w * _L,), jnp.float32),
            pltpu.VMEM((per_w * _ROWS,), jnp.int32),
            pltpu.VMEM((2, _CHUNK, dim), jnp.float32),
            pltpu.VMEM((2, _CHUNK), jnp.float32),
            pltpu.VMEM((2, _CHUNK), jnp.float32),
            pltpu.SemaphoreType.DMA,
            pltpu.SemaphoreType.DMA,
            pltpu.SemaphoreType.DMA,
            pltpu.SemaphoreType.DMA,
            pltpu.SemaphoreType.DMA,
        ],
    )
    def k(ent_hbm, hidx_hbm, rel_hbm, ridx_hbm, gidx_hbm,
          s_out, ss_out, qq_out,
          hidx_v, ridx_v, hrows_v, rrows_v, q_tab, hh_v, qq_v,
          gidx_v, rows_v, a_v, ss_v, sem,
          gsem0, gsem1, osem0, osem1):
        wid = lax.axis_index("s") * _NC + lax.axis_index("c")
        ebase = wid * per_w
        lanes = lax.iota(jnp.int32, _L)
        perms = [lanes ^ k for k in (8, 4, 2, 1)]

        pltpu.sync_copy(hidx_hbm.at[pl.ds(ebase, per_w)], hidx_v)
        pltpu.sync_copy(ridx_hbm.at[pl.ds(ebase, per_w)], ridx_v)
        pltpu.async_copy(ent_hbm.at[hidx_v], hrows_v, sem).wait()
        pltpu.async_copy(rel_hbm.at[ridx_v], rrows_v, sem).wait()

        def stage_a(e, carry):
            hch = [hrows_v[e, pl.ds(c * _L, _L)] for c in range(dc)]
            rch = [rrows_v[e, pl.ds(c * _L, _L)] for c in range(dc)]
            hh = _lane_sum(_tree_sum([h * h for h in hch]), perms)
            hr = _lane_sum(_tree_sum([h * r for h, r in zip(hch, rch)]),
                           perms)
            hh_s = hh[0]
            i = lax.bitcast_convert_type(hh_s, jnp.int32)
            i = 0x5F3759DF - lax.shift_right_logical(i, 1)
            y = lax.bitcast_convert_type(i, jnp.float32)
            for _ in range(3):
                y = y * (1.5 - 0.5 * hh_s * y * y)
            for c in range(dc):
                q_tab[e, pl.ds(c * _L, _L)] = hch[c] * y + rch[c]
            qq_v[pl.ds(e * _L, _L)] = 2.0 + 2.0 * hr * y
            return carry

        lax.fori_loop(0, per_w, stage_a, 0)
        pltpu.sync_copy(qq_v, qq_out.at[pl.ds(ebase * _L, per_w * _L)])

        pltpu.sync_copy(gidx_hbm.at[pl.ds(ebase * _ROWS, per_w * _ROWS)],
                        gidx_v)

        def gather_of(ch, p, gsem):
            return pltpu.async_copy(
                ent_hbm.at[gidx_v.at[pl.ds(ch * _CHUNK, _CHUNK)]],
                rows_v.at[p], gsem)

        def compute_chunk(ch, p):
            for sub in range(_CPE):
                e = ch * _CPE + sub
                qch = [q_tab[e, pl.ds(c * _L, _L)] for c in range(dc)]

                def group_body(g, carry2):
                    res_s = jnp.zeros((_L,), jnp.float32)
                    res_ss = jnp.zeros((_L,), jnp.float32)
                    for l in range(_L):
                        row = sub * _ROWS + g * _L + l
                        nch = [rows_v[p, row, pl.ds(c * _L, _L)]
                               for c in range(dc)]
                        sv = _lane_sum(
                            _tree_sum([n * q for n, q in zip(nch, qch)]),
                            perms)
                        ssv = _lane_sum(_tree_sum([n * n for n in nch]),
                                        perms)
                        res_s = jnp.where(lanes == l, sv, res_s)
                        res_ss = jnp.where(lanes == l, ssv, res_ss)
                    off = sub * _ROWS + g * _L
                    a_v[p, pl.ds(off, _L)] = res_s
                    ss_v[p, pl.ds(off, _L)] = res_ss
                    return carry2

                lax.fori_loop(0, _ROWS // _L, group_body, 0)

        def writeout(ch, p, osem):
            rbase = (ebase + ch * _CPE) * _ROWS
            pltpu.async_copy(a_v.at[p], s_out.at[pl.ds(rbase, _CHUNK)], osem)
            pltpu.async_copy(ss_v.at[p], ss_out.at[pl.ds(rbase, _CHUNK)],
                             osem)

        def drain_out(ch, p, osem):
            rbase = (ebase + ch * _CPE) * _ROWS
            pltpu.make_async_copy(
                a_v.at[p], s_out.at[pl.ds(rbase, _CHUNK)], osem).wait()
            pltpu.make_async_copy(
                ss_v.at[p], ss_out.at[pl.ds(rbase, _CHUNK)], osem).wait()

        gather_of(0, 0, gsem0)

        def pair_body(i, carry):
            c0 = 2 * i
            pltpu.make_async_copy(
                ent_hbm.at[gidx_v.at[pl.ds(c0 * _CHUNK, _CHUNK)]],
                rows_v.at[0], gsem0).wait()
            gather_of(c0 + 1, 1, gsem1)

            @pl.when(i > 0)
            def _():
                drain_out(c0 - 2, 0, osem0)

            compute_chunk(c0, 0)
            writeout(c0, 0, osem0)

            pltpu.make_async_copy(
                ent_hbm.at[gidx_v.at[pl.ds((c0 + 1) * _CHUNK, _CHUNK)]],
                rows_v.at[1], gsem1).wait()
            gather_of((c0 + 2) % n_chunks, 0, gsem0)

            @pl.when(i > 0)
            def _():
                drain_out(c0 - 1, 1, osem1)

            compute_chunk(c0 + 1, 1)
            writeout(c0 + 1, 1, osem1)
            return carry

        lax.fori_loop(0, n_chunks // 2, pair_body, 0)
        pltpu.make_async_copy(
            ent_hbm.at[gidx_v.at[pl.ds(0, _CHUNK)]],
            rows_v.at[0], gsem0).wait()
        drain_out(n_chunks - 2, 0, osem0)
        drain_out(n_chunks - 1, 1, osem1)

    return k


def _tc_epilogue_body(s_ref, ss_ref, qq_ref, pos_ref, negs_ref):
    s = s_ref[...]
    ss = ss_ref[...]
    qq = qq_ref[...][:, :1]
    rn = 1.0 / jnp.maximum(jnp.sqrt(ss), 1e-12)
    sc2 = jnp.maximum(qq + 1.0 - 2.0 * s * rn, 0.0)
    sc = -jnp.sqrt(sc2)
    pos_ref[...] = sc[:, 0]
    negs_ref[...] = sc[:, 1:51]


def _tc_epilogue(s, ss, qq, batch, n_neg):
    blk = 512
    return pl.pallas_call(
        _tc_epilogue_body,
        grid=(batch // blk,),
        in_specs=[
            pl.BlockSpec((blk, _ROWS), lambda i: (i, 0)),
            pl.BlockSpec((blk, _ROWS), lambda i: (i, 0)),
            pl.BlockSpec((blk, _L), lambda i: (i, 0)),
        ],
        out_specs=[
            pl.BlockSpec((blk,), lambda i: (i,)),
            pl.BlockSpec((blk, n_neg), lambda i: (i, 0)),
        ],
        out_shape=[
            jax.ShapeDtypeStruct((batch,), jnp.float32),
            jax.ShapeDtypeStruct((batch, n_neg), jnp.float32),
        ],
    )(s, ss, qq)


def kernel(head_idx, relation_idx, tail_idx, negative_idx, entity_emb, relation_emb):
    batch = head_idx.shape[0]
    n_neg = negative_idx.shape[1]
    dim = entity_emb.shape[1]
    gidx = jnp.concatenate(
        [tail_idx[:, None], negative_idx, negative_idx[:, : _ROWS - 1 - n_neg]],
        axis=1).reshape(-1)
    sc = _sc_build(batch, dim)
    s, ss, qq = sc(entity_emb, head_idx, relation_emb, relation_idx, gidx)
    pos, negs = _tc_epilogue(
        s.reshape(batch, _ROWS), ss.reshape(batch, _ROWS),
        qq.reshape(batch, _L), batch, n_neg)
    return pos, negs
